# baseline (speedup 1.0000x reference)
import jax
import jax.numpy as jnp
from jax.experimental import pallas as pl
from jax.experimental.pallas import tpu as pltpu

_TILE = 512
_R = 16


def _vq_kernel(x_ref, c_ref, o_ref, cb_ref, c2_ref):
    @pl.when(pl.program_id(0) == 0)
    def _():
        c = c_ref[...]
        cb_ref[...] = c.astype(jnp.bfloat16)
        c2 = jnp.sum(c * c, axis=1, keepdims=True)
        c2_ref[...] = jnp.broadcast_to(c2, c2_ref.shape)

    xm2 = (x_ref[...] * (-2.0)).astype(jnp.bfloat16)
    dots = jax.lax.dot_general(
        cb_ref[...], xm2,
        dimension_numbers=(((1,), (1,)), ((), ())),
        preferred_element_type=jnp.float32,
    )

    n_chunks = dots.shape[0] // _R
    n_lt = _TILE // 128

    c2c = c2_ref[:_R, :]
    v = [dots[:_R, t * 128:(t + 1) * 128] + c2c for t in range(n_lt)]
    cid = [jnp.zeros((_R, 128), jnp.int32) for _ in range(n_lt)]
    for j in range(1, n_chunks):
        c2c = c2_ref[j * _R:(j + 1) * _R, :]
        for t in range(n_lt):
            s = dots[j * _R:(j + 1) * _R, t * 128:(t + 1) * 128] + c2c
            upd = s < v[t]
            v[t] = jnp.where(upd, s, v[t])
            cid[t] = jnp.where(upd, jnp.int32(j), cid[t])

    outs = []
    for t in range(n_lt):
        vt = v[t]
        it = cid[t] * _R + jax.lax.broadcasted_iota(jnp.int32, vt.shape, 0)
        r = _R
        while r > 8:
            h = r // 2
            va, vb = vt[:h, :], vt[h:, :]
            ia, ib = it[:h, :], it[h:, :]
            take_b = (vb < va) | ((vb == va) & (ib < ia))
            vt = jnp.where(take_b, vb, va)
            it = jnp.where(take_b, ib, ia)
            r = h
        for sh in (1, 2, 4):
            vr = pltpu.roll(vt, sh, axis=0)
            ir = pltpu.roll(it, sh, axis=0)
            take_r = (vr < vt) | ((vr == vt) & (ir < it))
            vt = jnp.where(take_r, vr, vt)
            it = jnp.where(take_r, ir, it)
        outs.append(it[:1, :])

    o_ref[...] = jnp.concatenate(outs, axis=1).reshape(1, 1, _TILE)


def kernel(inp, clusters):
    B, T, D = inp.shape
    C = clusters.shape[0]
    x = inp.reshape(B * T, D)
    nt = (B * T) // _TILE
    out = pl.pallas_call(
        _vq_kernel,
        grid=(nt,),
        in_specs=[
            pl.BlockSpec((_TILE, D), lambda i: (i, 0)),
            pl.BlockSpec((C, D), lambda i: (0, 0)),
        ],
        out_specs=pl.BlockSpec((1, 1, _TILE), lambda i: (i, 0, 0)),
        out_shape=jax.ShapeDtypeStruct((nt, 1, _TILE), jnp.int32),
        scratch_shapes=[
            pltpu.VMEM((C, D), jnp.bfloat16),
            pltpu.VMEM((C, 128), jnp.float32),
        ],
    )(x, clusters)
    return out.reshape(B, T)


# speedup vs baseline: 1.0041x; 1.0041x over previous
import jax
import jax.numpy as jnp
from jax.experimental import pallas as pl
from jax.experimental.pallas import tpu as pltpu

_TILE = 512
_R = 16


def _vq_kernel(x_ref, c_ref, o_ref, cb_ref, c2_ref):
    @pl.when(pl.program_id(0) == 0)
    def _():
        c = c_ref[...]
        cb_ref[...] = (c * (-2.0)).astype(jnp.bfloat16)
        c2 = jnp.sum(c * c, axis=1, keepdims=True)
        c2_ref[...] = jnp.broadcast_to(c2, c2_ref.shape)

    xm2 = x_ref[...].astype(jnp.bfloat16)
    dots = jax.lax.dot_general(
        cb_ref[...], xm2,
        dimension_numbers=(((1,), (1,)), ((), ())),
        preferred_element_type=jnp.float32,
    )

    n_chunks = dots.shape[0] // _R
    n_lt = _TILE // 128

    c2c = c2_ref[:_R, :]
    v = [dots[:_R, t * 128:(t + 1) * 128] + c2c for t in range(n_lt)]
    cid = [jnp.zeros((_R, 128), jnp.int32) for _ in range(n_lt)]
    for j in range(1, n_chunks):
        c2c = c2_ref[j * _R:(j + 1) * _R, :]
        for t in range(n_lt):
            s = dots[j * _R:(j + 1) * _R, t * 128:(t + 1) * 128] + c2c
            upd = s < v[t]
            v[t] = jnp.where(upd, s, v[t])
            cid[t] = jnp.where(upd, jnp.int32(j), cid[t])

    outs = []
    for t in range(n_lt):
        vt = v[t]
        it = cid[t] * _R + jax.lax.broadcasted_iota(jnp.int32, vt.shape, 0)
        r = _R
        while r > 8:
            h = r // 2
            va, vb = vt[:h, :], vt[h:, :]
            ia, ib = it[:h, :], it[h:, :]
            take_b = (vb < va) | ((vb == va) & (ib < ia))
            vt = jnp.where(take_b, vb, va)
            it = jnp.where(take_b, ib, ia)
            r = h
        for sh in (1, 2, 4):
            vr = pltpu.roll(vt, sh, axis=0)
            ir = pltpu.roll(it, sh, axis=0)
            take_r = (vr < vt) | ((vr == vt) & (ir < it))
            vt = jnp.where(take_r, vr, vt)
            it = jnp.where(take_r, ir, it)
        outs.append(it[:1, :])

    o_ref[...] = jnp.concatenate(outs, axis=1).reshape(1, 1, _TILE)


def kernel(inp, clusters):
    B, T, D = inp.shape
    C = clusters.shape[0]
    x = inp.reshape(B * T, D)
    nt = (B * T) // _TILE
    out = pl.pallas_call(
        _vq_kernel,
        grid=(nt,),
        in_specs=[
            pl.BlockSpec((_TILE, D), lambda i: (i, 0)),
            pl.BlockSpec((C, D), lambda i: (0, 0)),
        ],
        out_specs=pl.BlockSpec((1, 1, _TILE), lambda i: (i, 0, 0)),
        out_shape=jax.ShapeDtypeStruct((nt, 1, _TILE), jnp.int32),
        scratch_shapes=[
            pltpu.VMEM((C, D), jnp.bfloat16),
            pltpu.VMEM((C, 128), jnp.float32),
        ],
    )(x, clusters)
    return out.reshape(B, T)
